# baseline (speedup 1.0000x reference)
import jax
import jax.numpy as jnp
from jax.experimental import pallas as pl
from jax.experimental.pallas import tpu as pltpu

_ETA = 0.8
_CPR = 128
_LANES = 4 * _CPR


def _body(g_ref, a_ref, b_ref, p_ref, q_ref, o_ref):
    x = q_ref[...]
    a = a_ref[...].astype(jnp.float32)
    b = b_ref[...].astype(jnp.float32)
    prof = p_ref[...]

    ci = jax.lax.broadcasted_iota(jnp.int32, (_CPR, _LANES), 0)
    li = jax.lax.broadcasted_iota(jnp.int32, (_CPR, _LANES), 1)
    dil = (li // 4 == ci).astype(jnp.float32)

    t = 2.0 * a + b
    te = jax.lax.dot(t, dil, precision=jax.lax.Precision.HIGHEST,
                     preferred_element_type=jnp.float32)
    pe = jax.lax.dot(prof, dil, precision=jax.lax.Precision.HIGHEST,
                     preferred_element_type=jnp.float32)

    pos = jax.lax.broadcasted_iota(jnp.int32, x.shape, 1) & 3
    pm = jnp.maximum(x, pltpu.roll(x, _LANES - 1, 1))
    mv = jnp.where(
        pos == 0, pm,
        jnp.where(pos == 1, pltpu.roll(pm, _LANES - 1, 1),
                  jnp.where(pos == 2, pltpu.roll(pm, 2, 1), pltpu.roll(pm, 1, 1))))

    ge = g_ref[0, 0]
    upd = (1.0 - _ETA) * x + _ETA * pe + ge * mv
    o_ref[...] = jnp.where(te == pos.astype(jnp.float32), upd, x)


def kernel(alpha, gamma, type_t_matrix, type_t1_matrix, Q_tensor, profit_matrix):
    n = Q_tensor.shape[0]
    r = n // _CPR
    br = 256
    while r % br:
        br //= 2
    grid = r // br

    qv = Q_tensor.reshape(r, _LANES)
    av = type_t_matrix.reshape(r, _CPR)
    bv = type_t1_matrix.reshape(r, _CPR)
    pv = profit_matrix.reshape(r, _CPR)
    gv = jnp.full((8, 128), jnp.float32(gamma) * _ETA, dtype=jnp.float32)

    out = pl.pallas_call(
        _body,
        grid=(grid,),
        in_specs=[
            pl.BlockSpec((8, 128), lambda i: (0, 0)),
            pl.BlockSpec((br, _CPR), lambda i: (i, 0)),
            pl.BlockSpec((br, _CPR), lambda i: (i, 0)),
            pl.BlockSpec((br, _CPR), lambda i: (i, 0)),
            pl.BlockSpec((br, _LANES), lambda i: (i, 0)),
        ],
        out_specs=pl.BlockSpec((br, _LANES), lambda i: (i, 0)),
        out_shape=jax.ShapeDtypeStruct((r, _LANES), jnp.float32),
        compiler_params=pltpu.CompilerParams(
            dimension_semantics=("parallel",)),
    )(gv, av, bv, pv, qv)
    return out.reshape(Q_tensor.shape)


# speedup vs baseline: 163.5304x; 163.5304x over previous
import jax
import jax.numpy as jnp
from jax.experimental import pallas as pl
from jax.experimental.pallas import tpu as pltpu

_ETA = 0.8
_BA = 8
_L = 4096


def _body(g_ref, a_ref, b_ref, p_ref, q_ref, o_ref):
    bh = q_ref.shape[1]
    q00 = q_ref[0, :, 0, :]
    q01 = q_ref[0, :, 1, :]
    q10 = q_ref[1, :, 0, :]
    q11 = q_ref[1, :, 1, :]
    a = a_ref[...].reshape(bh, 128)
    b = b_ref[...].reshape(bh, 128)
    prof = p_ref[...].reshape(bh, 128)

    az = a == 0
    bz = b == 0
    mv = jnp.where(bz, jnp.maximum(q00, q01), jnp.maximum(q10, q11))
    old = jnp.where(az, jnp.where(bz, q00, q01), jnp.where(bz, q10, q11))
    ge = g_ref[0, 0]
    upd = (1.0 - _ETA) * old + _ETA * prof + ge * mv
    o_ref[0, :, 0, :] = jnp.where(az & bz, upd, q00)
    o_ref[0, :, 1, :] = jnp.where(az & ~bz, upd, q01)
    o_ref[1, :, 0, :] = jnp.where(~az & bz, upd, q10)
    o_ref[1, :, 1, :] = jnp.where(~az & ~bz, upd, q11)


def kernel(alpha, gamma, type_t_matrix, type_t1_matrix, Q_tensor, profit_matrix):
    n = Q_tensor.shape[0]
    l = type_t_matrix.shape[0]
    nh = n // 128
    bh = _BA * l // 128
    grid = l // _BA

    x = jnp.transpose(Q_tensor, (1, 0, 2)).reshape(2, nh, 128, 2)
    x = jnp.transpose(x, (0, 1, 3, 2))
    gv = jnp.full((8, 128), jnp.float32(gamma) * _ETA, dtype=jnp.float32)

    spec2d = pl.BlockSpec((_BA, l), lambda i: (i, 0))
    qspec = pl.BlockSpec((2, bh, 2, 128), lambda i: (0, i, 0, 0))

    out4 = pl.pallas_call(
        _body,
        grid=(grid,),
        in_specs=[
            pl.BlockSpec((8, 128), lambda i: (0, 0)),
            spec2d,
            spec2d,
            spec2d,
            qspec,
        ],
        out_specs=qspec,
        out_shape=jax.ShapeDtypeStruct((2, nh, 2, 128), jnp.float32),
        compiler_params=pltpu.CompilerParams(
            dimension_semantics=("parallel",)),
    )(gv, type_t_matrix, type_t1_matrix, profit_matrix, x)

    out = jnp.transpose(out4, (0, 1, 3, 2)).reshape(2, n, 2)
    return jnp.transpose(out, (1, 0, 2))
